# baseline (speedup 1.0000x reference)
import functools

import jax
import jax.numpy as jnp
from jax import lax
from jax.experimental import pallas as pl
from jax.experimental.pallas import tpu as pltpu
from jax.experimental.pallas import tpu_sc as plsc

E = 64
NW = 32
CH = 128
K = 10
CHUNK = CH * K


def _gather_sc(table, idx):
    N = idx.shape[0]
    b_per_w = N // NW
    n_chunks = b_per_w // CHUNK
    mesh = plsc.VectorSubcoreMesh(core_axis_name="c", subcore_axis_name="s")

    @functools.partial(
        pl.kernel,
        mesh=mesh,
        out_type=jax.ShapeDtypeStruct((N, E), jnp.float32),
        compiler_params=pltpu.CompilerParams(use_tc_tiling_on_sc=False),
        scratch_types=[
            pltpu.VMEM((CHUNK,), jnp.int32),
            pltpu.VMEM((CHUNK, E), jnp.float32),
            pltpu.SemaphoreType.DMA,
        ],
    )
    def k(table_hbm, idx_hbm, out_hbm, idx_v, rows_v, sem):
        c = lax.axis_index("c")
        s = lax.axis_index("s")
        wid = s * 2 + c
        base = wid * b_per_w

        def body(j, carry):
            off = base + j * CHUNK
            pltpu.sync_copy(idx_hbm.at[pl.ds(off, CHUNK)], idx_v)
            copies = []
            for t in range(K):
                copies.append(
                    pltpu.async_copy(
                        table_hbm.at[idx_v.at[pl.ds(t * CH, CH)]],
                        rows_v.at[pl.ds(t * CH, CH)],
                        sem,
                    )
                )
            for cp in copies:
                cp.wait()
            pltpu.sync_copy(rows_v, out_hbm.at[pl.ds(off, CHUNK)])
            return carry

        lax.fori_loop(0, n_chunks, body, 0)

    return k(table, idx)


def _matmul_tc(x, w):
    N = x.shape[0]
    BLK = 2048
    grid = N // BLK

    def body(x_ref, w_ref, o_ref):
        o_ref[...] = lax.dot_general(
            x_ref[...], w_ref[...], (((1,), (1,)), ((), ())),
            preferred_element_type=jnp.float32,
        )

    return pl.pallas_call(
        body,
        grid=(grid,),
        in_specs=[
            pl.BlockSpec((BLK, E), lambda i: (i, 0)),
            pl.BlockSpec((E, E), lambda i: (0, 0)),
        ],
        out_specs=pl.BlockSpec((BLK, E), lambda i: (i, 0)),
        out_shape=jax.ShapeDtypeStruct((N, E), jnp.float32),
    )(x, w)


def kernel(sent1, sent2, embedding_table, W):
    B, S = sent1.shape
    n = B * S
    idx = jnp.concatenate(
        [sent1.reshape(-1), sent2.reshape(-1)]
    ).astype(jnp.int32)
    gathered = _gather_sc(embedding_table, idx)
    y = _matmul_tc(gathered, W)
    s1 = y[:n].reshape(B, S, E)
    s2 = y[n:].reshape(B, S, E)
    return (s1, s2)
